# baseline (speedup 1.0000x reference)
import functools

import jax
import jax.numpy as jnp
from jax import lax
from jax.experimental import pallas as pl
from jax.experimental.pallas import tpu as pltpu
from jax.experimental.pallas import tpu_sc as plsc

VOCAB = 1_000_000
EMB = 64
ROWS = 4096
COLS = 200
CHUNK = 128
TB = ROWS // CHUNK

_info = plsc.get_sparse_core_info()
NC = _info.num_cores
NS = _info.num_subcores
NW = NC * NS

_mesh = plsc.VectorSubcoreMesh(core_axis_name="c", subcore_axis_name="s")


@functools.partial(
    pl.kernel,
    out_type=jax.ShapeDtypeStruct((COLS, 8, TB, 8, CHUNK), jnp.float32),
    mesh=_mesh,
    scratch_types=[
        pltpu.VMEM((COLS, CHUNK), jnp.int32),
        pltpu.VMEM((CHUNK, EMB), jnp.float32),
        pltpu.VMEM((CHUNK, EMB), jnp.float32),
        pltpu.VMEM((EMB, CHUNK + 1), jnp.float32),
        pltpu.VMEM((EMB, CHUNK + 1), jnp.float32),
        pltpu.SemaphoreType.DMA((2,)),
        pltpu.SemaphoreType.DMA((2,)),
    ],
    compiler_params=pltpu.CompilerParams(use_tc_tiling_on_sc=False,
                                         needs_layout_passes=False),
)
def _embed_sc(idx_hbm, table_hbm, out_hbm, idx_v, rows0, rows1, tr0, tr1,
              gsem, ssem):
    wid = lax.axis_index("s") * NC + lax.axis_index("c")
    rows = (rows0, rows1)
    trans = (tr0, tr1)

    b0 = pl.multiple_of(wid * CHUNK, CHUNK)
    pltpu.sync_copy(idx_hbm.at[:, pl.ds(b0, CHUNK)], idx_v)

    def fire_gather(t, b):
        pltpu.async_copy(table_hbm.at[idx_v.at[t]], rows[b], gsem.at[b])

    def wait_gather(b):
        pltpu.make_async_copy(table_hbm.at[pl.ds(0, CHUNK)], rows[b],
                              gsem.at[b]).wait()

    def wait_store(b):
        for te in range(8):
            pltpu.make_async_copy(trans[b].at[pl.ds(te * 8, 8),
                                              pl.ds(0, CHUNK)],
                                  out_hbm.at[0, te, 0], ssem.at[b]).wait()

    lane = lax.iota(jnp.int32, 16)
    e_vecs = [16 * k + lane for k in range(4)]

    def transpose_and_store(t, b):
        for c in range(CHUNK):
            cvec = jnp.full((16,), c, jnp.int32)
            vs = [rows[b][c, pl.ds(16 * k, 16)] for k in range(4)]
            for k in range(4):
                plsc.store_scatter(trans[b], [e_vecs[k], cvec], vs[k])
        for te in range(8):
            pltpu.async_copy(trans[b].at[pl.ds(te * 8, 8), pl.ds(0, CHUNK)],
                             out_hbm.at[t, te, wid], ssem.at[b])

    fire_gather(0, 0)
    fire_gather(1, 1)

    n_pairs = COLS // 2

    def pair_body(tt, carry):
        t0 = tt * 2
        for b in range(2):
            t = t0 + b
            wait_gather(b)

            @pl.when(tt > 0)
            def _():
                wait_store(b)

            transpose_and_store(t, b)

            @pl.when(tt < n_pairs - 1)
            def _():
                fire_gather(t + 2, b)

        return carry

    lax.fori_loop(0, n_pairs, pair_body, 0)
    wait_store(0)
    wait_store(1)


def kernel(input_tensor, weight):
    idx_t = input_tensor.astype(jnp.int32).swapaxes(0, 1)
    out5 = _embed_sc(idx_t, weight)
    return out5.transpose(2, 4, 0, 1, 3).reshape(ROWS, COLS, EMB)


# speedup vs baseline: 1.0128x; 1.0128x over previous
import functools

import jax
import jax.numpy as jnp
from jax import lax
from jax.experimental import pallas as pl
from jax.experimental.pallas import tpu as pltpu
from jax.experimental.pallas import tpu_sc as plsc

VOCAB = 1_000_000
EMB = 64
ROWS = 4096
COLS = 200
CHUNK = 128
TB = ROWS // CHUNK

_info = plsc.get_sparse_core_info()
NC = _info.num_cores
NS = _info.num_subcores
NW = NC * NS

_mesh = plsc.VectorSubcoreMesh(core_axis_name="c", subcore_axis_name="s")


@functools.partial(
    pl.kernel,
    out_type=jax.ShapeDtypeStruct((COLS, 8, TB, 8, CHUNK), jnp.float32),
    mesh=_mesh,
    scratch_types=[
        pltpu.VMEM((COLS, CHUNK), jnp.int32),
        pltpu.VMEM((CHUNK, EMB), jnp.float32),
        pltpu.VMEM((CHUNK, EMB), jnp.float32),
        pltpu.VMEM((8, 8, CHUNK + 1), jnp.float32),
        pltpu.VMEM((8, 8, CHUNK + 1), jnp.float32),
        pltpu.SemaphoreType.DMA((2,)),
        pltpu.SemaphoreType.DMA((2,)),
    ],
    compiler_params=pltpu.CompilerParams(use_tc_tiling_on_sc=False,
                                         needs_layout_passes=False),
)
def _embed_sc(idx_hbm, table_hbm, out_hbm, idx_v, rows0, rows1, tr0, tr1,
              gsem, ssem):
    wid = lax.axis_index("s") * NC + lax.axis_index("c")
    rows = (rows0, rows1)
    trans = (tr0, tr1)

    b0 = pl.multiple_of(wid * CHUNK, CHUNK)
    pltpu.sync_copy(idx_hbm.at[:, pl.ds(b0, CHUNK)], idx_v)

    def fire_gather(t, b):
        pltpu.async_copy(table_hbm.at[idx_v.at[t]], rows[b], gsem.at[b])

    def wait_gather(b):
        pltpu.make_async_copy(table_hbm.at[pl.ds(0, CHUNK)], rows[b],
                              gsem.at[b]).wait()

    def wait_store(b):
        pltpu.make_async_copy(trans[b].at[:, :, pl.ds(0, CHUNK)],
                              out_hbm.at[0, :, 0], ssem.at[b]).wait()

    lane = lax.iota(jnp.int32, 16)
    te_vecs = [(16 * k + lane) // 8 for k in range(4)]
    er_vec = lane % 8

    def transpose_and_store(t, b):
        for c in range(CHUNK):
            cvec = jnp.full((16,), c, jnp.int32)
            vs = [rows[b][c, pl.ds(16 * k, 16)] for k in range(4)]
            for k in range(4):
                plsc.store_scatter(trans[b], [te_vecs[k], er_vec, cvec],
                                   vs[k])
        for te in range(8):
            pltpu.async_copy(trans[b].at[te, :, pl.ds(0, CHUNK)],
                             out_hbm.at[t, te, wid], ssem.at[b])

    fire_gather(0, 0)
    fire_gather(1, 1)

    n_pairs = COLS // 2

    def pair_body(tt, carry):
        t0 = tt * 2
        for b in range(2):
            t = t0 + b
            wait_gather(b)

            @pl.when(tt > 0)
            def _():
                wait_store(b)

            transpose_and_store(t, b)

            @pl.when(tt < n_pairs - 1)
            def _():
                fire_gather(t + 2, b)

        return carry

    lax.fori_loop(0, n_pairs, pair_body, 0)
    wait_store(0)
    wait_store(1)


def kernel(input_tensor, weight):
    idx_t = input_tensor.astype(jnp.int32).swapaxes(0, 1)
    out5 = _embed_sc(idx_t, weight)
    return out5.transpose(2, 4, 0, 1, 3).reshape(ROWS, COLS, EMB)
